# Initial kernel scaffold; baseline (speedup 1.0000x reference)
import jax
import jax.numpy as jnp
from jax.experimental import pallas as pl


def kernel(seq_len, weight):
    raise NotImplementedError("write your pallas kernel here")


# speedup vs baseline: 42.5949x; 42.5949x over previous
import functools

import jax
import jax.numpy as jnp
from jax import lax
from jax.experimental import pallas as pl
from jax.experimental.pallas import tpu as pltpu
from jax.experimental.pallas import tpu_sc as plsc

MAXD = 128
NBUK = 2 * MAXD + 1
H = 16
S = 2048
L = 16
NC, NS = 2, 16
R = 8
PLEN = 4096
ROWS_PER_W = (H * S) // (NC * NS)
BLOCKS_PER_W = ROWS_PER_W // R
W_PIPE = 4


def _body(weight_hbm, out_hbm, w_vmem, p_scr, sem):
    cid = lax.axis_index("c")
    sid = lax.axis_index("s")
    wid = sid * NC + cid
    h = wid // 2
    half = wid % 2

    pltpu.sync_copy(weight_hbm, w_vmem)

    lane = lax.iota(jnp.int32, L)
    h_vec = jnp.full((L,), h, dtype=jnp.int32)

    for d in range(R):
        off = 7 - d - (S - 1)

        def fill(ci, carry, off=off):
            k = ci * L + lane + off
            b = jnp.clip(k, -MAXD, MAXD) + MAXD
            vals = plsc.load_gather(w_vmem, [b, h_vec])
            p_scr[d, pl.ds(ci * L, L)] = vals
            return carry

        lax.fori_loop(0, PLEN // L, fill, 0)

    def fire(t, carry):
        m = half * BLOCKS_PER_W + t
        start = pl.multiple_of((2040 - 8 * m) + 0 * t, 8)
        src = p_scr.at[:, pl.ds(start, S)]
        dst = out_hbm.at[h, pl.ds(8 * m, R)]
        cp = pltpu.async_copy(src, dst, sem)

        @pl.when(t >= W_PIPE)
        def _():
            cp.wait()

        return carry

    lax.fori_loop(0, BLOCKS_PER_W, fire, 0)

    for _ in range(W_PIPE):
        pltpu.make_async_copy(
            out_hbm.at[0, pl.ds(0, R)], p_scr.at[:, pl.ds(0, S)], sem
        ).wait()


_bias = functools.partial(
    pl.kernel,
    out_type=jax.ShapeDtypeStruct((H, S, S), jnp.float32),
    mesh=plsc.VectorSubcoreMesh(
        core_axis_name="c", subcore_axis_name="s", num_cores=NC, num_subcores=NS
    ),
    scratch_types=[
        pltpu.VMEM((NBUK, H), jnp.float32),
        pltpu.VMEM((R, PLEN), jnp.float32),
        pltpu.SemaphoreType.DMA,
    ],
    compiler_params=pltpu.CompilerParams(
        use_tc_tiling_on_sc=False, needs_layout_passes=False
    ),
)(_body)


def kernel(seq_len, weight):
    del seq_len
    return _bias(weight)


# speedup vs baseline: 43.0241x; 1.0101x over previous
import functools

import jax
import jax.numpy as jnp
from jax import lax
from jax.experimental import pallas as pl
from jax.experimental.pallas import tpu as pltpu
from jax.experimental.pallas import tpu_sc as plsc

MAXD = 128
NBUK = 2 * MAXD + 1
H = 16
S = 2048
L = 16
NC, NS = 2, 16
R = 16
PLEN = 4096
CHUNKS = PLEN // L
ROWS_PER_W = (H * S) // (NC * NS)
BLOCKS_PER_W = ROWS_PER_W // R
W_PIPE = 8

BAND_LO, BAND_HI = 119, 137
UNROLL = 7


def _body(weight_hbm, out_hbm, w_vmem, p_scr, sem):
    cid = lax.axis_index("c")
    sid = lax.axis_index("s")
    wid = sid * NC + cid
    h = wid // 2
    half = wid % 2

    pltpu.sync_copy(weight_hbm, w_vmem)

    lane = lax.iota(jnp.int32, L)
    h_vec = jnp.full((L,), h, dtype=jnp.int32)
    v_lo = plsc.load_gather(w_vmem, [jnp.zeros((L,), jnp.int32), h_vec])
    v_hi = plsc.load_gather(
        w_vmem, [jnp.full((L,), NBUK - 1, jnp.int32), h_vec]
    )

    for d in range(R):
        off = (R - 1) - d - (S - 1)

        def flanks(ci, carry, d=d):
            for u in range(UNROLL):
                c_lo = ci * UNROLL + u
                p_scr[d, pl.ds(c_lo * L, L)] = v_lo
                c_hi = BAND_HI + ci * UNROLL + u
                p_scr[d, pl.ds(c_hi * L, L)] = v_hi
            return carry

        lax.fori_loop(0, BAND_LO // UNROLL, flanks, 0)

        def band(ci, carry, d=d, off=off):
            k = ci * L + lane + off
            b = jnp.clip(k, -MAXD, MAXD) + MAXD
            p_scr[d, pl.ds(ci * L, L)] = plsc.load_gather(w_vmem, [b, h_vec])
            return carry

        lax.fori_loop(BAND_LO, BAND_HI, band, 0)

    def fire(t, carry):
        m = half * BLOCKS_PER_W + t
        start = pl.multiple_of((2032 - 16 * m) + 0 * t, 16)
        src = p_scr.at[:, pl.ds(start, S)]
        dst = out_hbm.at[h, pl.ds(R * m, R)]
        cp = pltpu.async_copy(src, dst, sem)

        @pl.when(t >= W_PIPE)
        def _():
            cp.wait()

        return carry

    lax.fori_loop(0, BLOCKS_PER_W, fire, 0)

    for _ in range(W_PIPE):
        pltpu.make_async_copy(
            out_hbm.at[0, pl.ds(0, R)], p_scr.at[:, pl.ds(0, S)], sem
        ).wait()


_bias = functools.partial(
    pl.kernel,
    out_type=jax.ShapeDtypeStruct((H, S, S), jnp.float32),
    mesh=plsc.VectorSubcoreMesh(
        core_axis_name="c", subcore_axis_name="s", num_cores=NC, num_subcores=NS
    ),
    scratch_types=[
        pltpu.VMEM((NBUK, H), jnp.float32),
        pltpu.VMEM((R, PLEN), jnp.float32),
        pltpu.SemaphoreType.DMA,
    ],
    compiler_params=pltpu.CompilerParams(
        use_tc_tiling_on_sc=False, needs_layout_passes=False
    ),
)(_body)


def kernel(seq_len, weight):
    del seq_len
    return _bias(weight)
